# baseline (speedup 1.0000x reference)
import jax
import jax.numpy as jnp
import numpy as np
from jax.experimental import pallas as pl

N = 10000
G = 100
NPG = 100
K = 16
HIDDEN = 128
FILTERS = 128
LAYERS = 3
NG = 50
CUTOFF = 10.0

_LOG2 = 0.6931471805599453
_STEP = CUTOFF / (NG - 1)
_SQC = (0.5 ** 0.5) / _STEP
_PI = 3.141592653589793


def _sp(x):
    return jnp.maximum(
        x, jnp.log(1.0 + jnp.exp(jnp.minimum(x, 30.0)))) - _LOG2


def _body(pos_ref, post_ref, z_ref, e16_ref, eyet_ref, emb_ref,
          mlp_w1_ref, mlp_b1_ref, mlp_w2_ref, mlp_b2_ref,
          conv_w1_ref, conv_w2_ref, conv_b2_ref,
          int_w_ref, int_b_ref,
          lin1_w_ref, lin1_b_ref, lin2_w_ref, lin2_b_ref,
          out_ref):
    p = pos_ref[0]
    pt = post_ref[0]
    zc = z_ref[0]

    d2 = jnp.zeros((NPG, NPG), jnp.float32)
    for c in range(3):
        dc = p[:, c:c + 1] - pt[c:c + 1, :]
        d2 = d2 + dc * dc
    row = jax.lax.broadcasted_iota(jnp.int32, (NPG, NPG), 0)
    col = jax.lax.broadcasted_iota(jnp.int32, (NPG, NPG), 1)
    inf = jnp.float32(float("inf"))
    cur = jnp.where(row == col, inf, d2)

    oh_parts, mv_parts = [], []
    for _ in range(K):
        mv = jnp.min(cur, axis=1, keepdims=True)
        cand = jnp.where(cur <= mv, col, NPG)
        jmin = jnp.min(cand, axis=1, keepdims=True)
        sel = col == jmin
        oh_parts.append(jnp.where(sel, 1.0, 0.0))
        mv_parts.append(mv)
        cur = jnp.where(sel, inf, cur)
    OH = jnp.concatenate(oh_parts, axis=0)
    D2 = jnp.concatenate(mv_parts, axis=1)

    valid = D2 <= CUTOFF * CUTOFF
    dist = jnp.sqrt(jnp.where(valid, D2, 1.0))
    CV = jnp.where(valid,
                   0.5 * jnp.cos(dist * (_PI / CUTOFF)) + 0.5, 0.0)

    ds = dist * _SQC
    dcol = jnp.concatenate([ds[:, k:k + 1] for k in range(K)], axis=0)
    db = jnp.dot(dcol, jnp.ones((1, NG), jnp.float32),
                 precision=jax.lax.Precision.HIGHEST,
                 preferred_element_type=jnp.float32)
    offs = jax.lax.broadcasted_iota(
        jnp.int32, (K * NPG, NG), 1).astype(jnp.float32) * (_STEP * _SQC)
    t = db - offs
    EA = jnp.exp(-(t * t))

    cvx = jnp.dot(CV, e16_ref[...],
                  precision=jax.lax.Precision.HIGHEST,
                  preferred_element_type=jnp.float32)
    SUMC = cvx * eyet_ref[...]

    ohz = jnp.where(zc == col, 1.0, 0.0)
    h = jnp.dot(ohz, emb_ref[...], precision=jax.lax.Precision.HIGHEST,
                preferred_element_type=jnp.float32)

    for l in range(LAYERS):
        A = _sp(jnp.dot(EA, mlp_w1_ref[l],
                        preferred_element_type=jnp.float32) + mlp_b1_ref[l])
        W = jnp.dot(A, mlp_w2_ref[l],
                    preferred_element_type=jnp.float32) + mlp_b2_ref[l]
        hx = jnp.dot(h, conv_w1_ref[l], preferred_element_type=jnp.float32)
        XJ = jnp.dot(OH, hx, precision=jax.lax.Precision.HIGHEST,
                     preferred_element_type=jnp.float32)
        P = XJ * W
        m = jnp.dot(SUMC, P, precision=jax.lax.Precision.HIGHEST,
                    preferred_element_type=jnp.float32)
        m = jnp.dot(m, conv_w2_ref[l],
                    preferred_element_type=jnp.float32) + conv_b2_ref[l]
        m = _sp(m)
        m = jnp.dot(m, int_w_ref[l],
                    preferred_element_type=jnp.float32) + int_b_ref[l]
        h = h + m

    t2 = _sp(jnp.dot(h, lin1_w_ref[...],
                     preferred_element_type=jnp.float32) + lin1_b_ref[...])
    y = jnp.dot(t2, lin2_w_ref[...],
                preferred_element_type=jnp.float32) + lin2_b_ref[...]
    s = jnp.sum(y)
    out_ref[...] = jnp.broadcast_to(s, (1, 1, 128))


def kernel(z, pos, batch, emb, mlp_w1, mlp_b1, mlp_w2, mlp_b2,
           conv_w1, conv_w2, conv_b2, int_w, int_b,
           lin1_w, lin1_b, lin2_w, lin2_b):
    del batch
    posg = pos.reshape(G, NPG, 3)
    post = jnp.swapaxes(posg, 1, 2)
    zg = z.reshape(G, NPG, 1).astype(jnp.int32)
    b1 = mlp_b1.reshape(LAYERS, 1, FILTERS)
    b2 = mlp_b2.reshape(LAYERS, 1, FILTERS)
    cb2 = conv_b2.reshape(LAYERS, 1, HIDDEN)
    ib = int_b.reshape(LAYERS, 1, HIDDEN)
    l1b = lin1_b.reshape(1, HIDDEN // 2)
    l2b = lin2_b.reshape(1, 1)
    e16 = jnp.asarray(np.kron(np.eye(K, dtype=np.float32),
                              np.ones((1, NPG), np.float32)))
    eyet = jnp.asarray(np.tile(np.eye(NPG, dtype=np.float32),
                               (1, K)))

    def full(shape):
        nd = len(shape)
        return pl.BlockSpec(shape, lambda g, _nd=nd: (0,) * _nd)

    out = pl.pallas_call(
        _body,
        grid=(G,),
        in_specs=[
            pl.BlockSpec((1, NPG, 3), lambda g: (g, 0, 0)),
            pl.BlockSpec((1, 3, NPG), lambda g: (g, 0, 0)),
            pl.BlockSpec((1, NPG, 1), lambda g: (g, 0, 0)),
            full((K, K * NPG)),
            full((NPG, K * NPG)),
            full((100, HIDDEN)),
            full((LAYERS, NG, FILTERS)),
            full((LAYERS, 1, FILTERS)),
            full((LAYERS, FILTERS, FILTERS)),
            full((LAYERS, 1, FILTERS)),
            full((LAYERS, HIDDEN, FILTERS)),
            full((LAYERS, FILTERS, HIDDEN)),
            full((LAYERS, 1, HIDDEN)),
            full((LAYERS, HIDDEN, HIDDEN)),
            full((LAYERS, 1, HIDDEN)),
            full((HIDDEN, HIDDEN // 2)),
            full((1, HIDDEN // 2)),
            full((HIDDEN // 2, 1)),
            full((1, 1)),
        ],
        out_specs=pl.BlockSpec((1, 1, 128), lambda g: (g, 0, 0)),
        out_shape=jax.ShapeDtypeStruct((G, 1, 128), jnp.float32),
    )(posg, post, zg, e16, eyet, emb, mlp_w1, b1, mlp_w2, b2,
      conv_w1, conv_w2, cb2, int_w, ib, lin1_w, l1b, lin2_w, l2b)
    return out[:, 0, 0]


# speedup vs baseline: 3.0771x; 3.0771x over previous
import jax
import jax.numpy as jnp
from jax.experimental import pallas as pl

N = 10000
G = 100
NPG = 100
K = 16
HIDDEN = 128
FILTERS = 128
LAYERS = 3
NG = 50
CUTOFF = 10.0

G_BLK = 4
B = G_BLK * NPG
E = K * B

_LOG2 = 0.6931471805599453
_STEP = CUTOFF / (NG - 1)
_SQC = (0.5 ** 0.5) / _STEP
_PI = 3.141592653589793
_HI = jax.lax.Precision.HIGHEST


def _sp(x):
    return jnp.maximum(
        x, jnp.log(1.0 + jnp.exp(jnp.minimum(x, 30.0)))) - _LOG2


def _body(pos_ref, post_ref, z_ref, emb_ref,
          mlp_w1_ref, mlp_b1_ref, mlp_w2_ref, mlp_b2_ref,
          conv_w1_ref, conv_w2_ref, conv_b2_ref,
          int_w_ref, int_b_ref,
          lin1_w_ref, lin1_b_ref, lin2_w_ref, lin2_b_ref,
          out_ref):
    inf = jnp.float32(float("inf"))
    eye = (jax.lax.broadcasted_iota(jnp.int32, (NPG, NPG), 0) ==
           jax.lax.broadcasted_iota(jnp.int32, (NPG, NPG), 1))
    blocks = []
    for b in range(G_BLK):
        p = pos_ref[b]
        pt = post_ref[b]
        d2 = jnp.zeros((NPG, NPG), jnp.float32)
        for c in range(3):
            dc = p[:, c:c + 1] - pt[c:c + 1, :]
            d2 = d2 + dc * dc
        blocks.append(jnp.where(eye, inf, d2))
    cur = jnp.concatenate(blocks, axis=0)

    col = jax.lax.broadcasted_iota(jnp.int32, (B, NPG), 1)

    oh_parts, mv_parts = [], []
    for _ in range(K):
        mv = jnp.min(cur, axis=1, keepdims=True)
        cand = jnp.where(cur <= mv, col, NPG)
        jmin = jnp.min(cand, axis=1, keepdims=True)
        sel = col == jmin
        oh_parts.append(jnp.where(sel, 1.0, 0.0))
        mv_parts.append(mv)
        cur = jnp.where(sel, inf, cur)
    D2 = jnp.concatenate(mv_parts, axis=1)
    OHs = [jnp.concatenate([oh_parts[k][b * NPG:(b + 1) * NPG]
                            for k in range(K)], axis=0)
           for b in range(G_BLK)]

    valid = D2 <= CUTOFF * CUTOFF
    dist = jnp.sqrt(jnp.where(valid, D2, 1.0))
    CV = jnp.where(valid,
                   0.5 * jnp.cos(dist * (_PI / CUTOFF)) + 0.5, 0.0)

    ds = dist * _SQC
    dcol = jnp.concatenate([ds[:, k:k + 1] for k in range(K)], axis=0)
    db = jnp.dot(dcol, jnp.ones((1, NG), jnp.float32),
                 precision=_HI, preferred_element_type=jnp.float32)
    offs = jax.lax.broadcasted_iota(
        jnp.int32, (E, NG), 1).astype(jnp.float32) * (_STEP * _SQC)
    t = db - offs
    EA = jnp.exp(-(t * t))

    cvcol = jnp.concatenate([CV[:, k:k + 1] for k in range(K)], axis=0)
    CVE = jnp.dot(cvcol, jnp.ones((1, FILTERS), jnp.float32),
                  precision=_HI, preferred_element_type=jnp.float32)

    zc = z_ref[...].reshape(B, 1)
    ohz = jnp.where(zc == col, 1.0, 0.0)
    h = jnp.dot(ohz, emb_ref[...], preferred_element_type=jnp.float32)

    for l in range(LAYERS):
        A = _sp(jnp.dot(EA, mlp_w1_ref[l],
                        preferred_element_type=jnp.float32) + mlp_b1_ref[l])
        W = (jnp.dot(A, mlp_w2_ref[l],
                     preferred_element_type=jnp.float32) + mlp_b2_ref[l]) * CVE
        hx = jnp.dot(h, conv_w1_ref[l], preferred_element_type=jnp.float32)
        ms = []
        for b in range(G_BLK):
            XJ = jnp.dot(OHs[b], hx[b * NPG:(b + 1) * NPG],
                         preferred_element_type=jnp.float32)
            m = XJ[0:NPG] * W[b * NPG:b * NPG + NPG]
            for k in range(1, K):
                m = m + (XJ[k * NPG:(k + 1) * NPG] *
                         W[k * B + b * NPG:k * B + b * NPG + NPG])
            ms.append(m)
        m = jnp.concatenate(ms, axis=0)
        m = jnp.dot(m, conv_w2_ref[l],
                    preferred_element_type=jnp.float32) + conv_b2_ref[l]
        m = _sp(m)
        m = jnp.dot(m, int_w_ref[l],
                    preferred_element_type=jnp.float32) + int_b_ref[l]
        h = h + m

    t2 = _sp(jnp.dot(h, lin1_w_ref[...],
                     preferred_element_type=jnp.float32) + lin1_b_ref[...])
    y = jnp.dot(t2, lin2_w_ref[...],
                preferred_element_type=jnp.float32) + lin2_b_ref[...]
    outs = jnp.concatenate(
        [jnp.broadcast_to(jnp.sum(y[b * NPG:(b + 1) * NPG]), (1, 128))
         for b in range(G_BLK)], axis=0)
    out_ref[0] = outs


def kernel(z, pos, batch, emb, mlp_w1, mlp_b1, mlp_w2, mlp_b2,
           conv_w1, conv_w2, conv_b2, int_w, int_b,
           lin1_w, lin1_b, lin2_w, lin2_b):
    del batch
    posg = pos.reshape(G, NPG, 3)
    post = jnp.swapaxes(posg, 1, 2)
    zg = z.reshape(G, NPG, 1).astype(jnp.int32)
    b1 = mlp_b1.reshape(LAYERS, 1, FILTERS)
    b2 = mlp_b2.reshape(LAYERS, 1, FILTERS)
    cb2 = conv_b2.reshape(LAYERS, 1, HIDDEN)
    ib = int_b.reshape(LAYERS, 1, HIDDEN)
    l1b = lin1_b.reshape(1, HIDDEN // 2)
    l2b = lin2_b.reshape(1, 1)

    def full(shape):
        nd = len(shape)
        return pl.BlockSpec(shape, lambda g, _nd=nd: (0,) * _nd)

    out = pl.pallas_call(
        _body,
        grid=(G // G_BLK,),
        in_specs=[
            pl.BlockSpec((G_BLK, NPG, 3), lambda g: (g, 0, 0)),
            pl.BlockSpec((G_BLK, 3, NPG), lambda g: (g, 0, 0)),
            pl.BlockSpec((G_BLK, NPG, 1), lambda g: (g, 0, 0)),
            full((100, HIDDEN)),
            full((LAYERS, NG, FILTERS)),
            full((LAYERS, 1, FILTERS)),
            full((LAYERS, FILTERS, FILTERS)),
            full((LAYERS, 1, FILTERS)),
            full((LAYERS, HIDDEN, FILTERS)),
            full((LAYERS, FILTERS, HIDDEN)),
            full((LAYERS, 1, HIDDEN)),
            full((LAYERS, HIDDEN, HIDDEN)),
            full((LAYERS, 1, HIDDEN)),
            full((HIDDEN, HIDDEN // 2)),
            full((1, HIDDEN // 2)),
            full((HIDDEN // 2, 1)),
            full((1, 1)),
        ],
        out_specs=pl.BlockSpec((1, G_BLK, 128), lambda g: (g, 0, 0)),
        out_shape=jax.ShapeDtypeStruct((G // G_BLK, G_BLK, 128), jnp.float32),
    )(posg, post, zg, emb, mlp_w1, b1, mlp_w2, b2,
      conv_w1, conv_w2, cb2, int_w, ib, lin1_w, l1b, lin2_w, l2b)
    return out[:, :, 0].reshape(G)


# speedup vs baseline: 3.7167x; 1.2078x over previous
import jax
import jax.numpy as jnp
import numpy as np
from jax.experimental import pallas as pl

N = 10000
G = 100
NPG = 100
K = 16
HIDDEN = 128
FILTERS = 128
LAYERS = 3
NG = 50
CUTOFF = 10.0

G_BLK = 4
R = G_BLK * NPG
E = K * R

_LOG2 = 0.6931471805599453
_STEP = CUTOFF / (NG - 1)
_SQC = (0.5 ** 0.5) / _STEP
_PI = 3.141592653589793
_HI = jax.lax.Precision.HIGHEST
_TL = (((0,), (0,)), ((), ()))


def _sp(x):
    return jnp.maximum(
        x, jnp.log(1.0 + jnp.exp(jnp.minimum(x, 30.0)))) - _LOG2


def _body(q_ref, prow_ref, z_ref, eyem_ref, emb_ref,
          mlp_w1_ref, mlp_b1_ref, mlp_w2_ref, mlp_b2_ref,
          conv_w1_ref, conv_w2_ref, conv_b2_ref,
          int_w_ref, int_b_ref,
          lin1_w_ref, lin1_b_ref, lin2_w_ref, lin2_b_ref,
          out_ref):
    inf = jnp.float32(float("inf"))
    q = q_ref[0]
    pr = prow_ref[0]

    d2 = jnp.zeros((NPG, R), jnp.float32)
    for c in range(3):
        dc = q[c] - pr[c:c + 1, :]
        d2 = d2 + dc * dc
    cur = jnp.where(eyem_ref[...] > 0.0, inf, d2)

    subi = jax.lax.broadcasted_iota(jnp.int32, (NPG, R), 0)

    sel_parts, mv_parts = [], []
    for _ in range(K):
        mv = jnp.min(cur, axis=0, keepdims=True)
        cand = jnp.where(cur <= mv, subi, NPG)
        jmin = jnp.min(cand, axis=0, keepdims=True)
        sel = subi == jmin
        sel_parts.append(jnp.where(sel, 1.0, 0.0))
        mv_parts.append(mv)
        cur = jnp.where(sel, inf, cur)
    D2 = jnp.concatenate(mv_parts, axis=0)

    valid = D2 <= CUTOFF * CUTOFF
    dist = jnp.sqrt(jnp.where(valid, D2, 1.0))
    CV = jnp.where(valid,
                   0.5 * jnp.cos(dist * (_PI / CUTOFF)) + 0.5, 0.0)

    ds = dist * _SQC
    ones_ng = jnp.ones((1, NG), jnp.float32)
    DB = jnp.concatenate(
        [jax.lax.dot_general(ds[k:k + 1], ones_ng, _TL, precision=_HI,
                             preferred_element_type=jnp.float32)
         for k in range(K)], axis=0)
    offs = jax.lax.broadcasted_iota(
        jnp.int32, (E, NG), 1).astype(jnp.float32) * (_STEP * _SQC)
    t = DB - offs
    EA = jnp.exp(-(t * t))

    ones_f = jnp.ones((1, FILTERS), jnp.float32)
    CVE = jnp.concatenate(
        [jax.lax.dot_general(CV[k:k + 1], ones_f, _TL, precision=_HI,
                             preferred_element_type=jnp.float32)
         for k in range(K)], axis=0)

    zr = z_ref[0]
    ohz = jnp.where(subi == zr, 1.0, 0.0)
    h = jax.lax.dot_general(ohz, emb_ref[...], _TL,
                            preferred_element_type=jnp.float32)

    selkb = [[sel_parts[k][:, b * NPG:(b + 1) * NPG] for b in range(G_BLK)]
             for k in range(K)]

    for l in range(LAYERS):
        A = _sp(jnp.dot(EA, mlp_w1_ref[l],
                        preferred_element_type=jnp.float32) + mlp_b1_ref[l])
        W = (jnp.dot(A, mlp_w2_ref[l],
                     preferred_element_type=jnp.float32) + mlp_b2_ref[l]) * CVE
        hx = jnp.dot(h, conv_w1_ref[l], preferred_element_type=jnp.float32)
        ms = []
        for b in range(G_BLK):
            hxb = hx[b * NPG:(b + 1) * NPG]
            m = None
            for k in range(K):
                XJ = jax.lax.dot_general(
                    selkb[k][b], hxb, _TL,
                    preferred_element_type=jnp.float32)
                p = XJ * W[k * R + b * NPG:k * R + b * NPG + NPG]
                m = p if m is None else m + p
            ms.append(m)
        m = jnp.concatenate(ms, axis=0)
        m = jnp.dot(m, conv_w2_ref[l],
                    preferred_element_type=jnp.float32) + conv_b2_ref[l]
        m = _sp(m)
        m = jnp.dot(m, int_w_ref[l],
                    preferred_element_type=jnp.float32) + int_b_ref[l]
        h = h + m

    t2 = _sp(jnp.dot(h, lin1_w_ref[...],
                     preferred_element_type=jnp.float32) + lin1_b_ref[...])
    y = jnp.dot(t2, lin2_w_ref[...],
                preferred_element_type=jnp.float32) + lin2_b_ref[...]
    outs = jnp.concatenate(
        [jnp.broadcast_to(jnp.sum(y[b * NPG:(b + 1) * NPG]), (1, 128))
         for b in range(G_BLK)], axis=0)
    out_ref[0] = outs


def kernel(z, pos, batch, emb, mlp_w1, mlp_b1, mlp_w2, mlp_b2,
           conv_w1, conv_w2, conv_b2, int_w, int_b,
           lin1_w, lin1_b, lin2_w, lin2_b):
    del batch
    NP = G // G_BLK
    posb = pos.reshape(NP, G_BLK, NPG, 3)
    q = jnp.broadcast_to(
        posb.transpose(0, 3, 2, 1)[:, :, :, :, None],
        (NP, 3, NPG, G_BLK, NPG)).reshape(NP, 3, NPG, R)
    prow = pos.reshape(NP, R, 3).transpose(0, 2, 1)
    zr = z.reshape(NP, 1, R).astype(jnp.int32)
    eyem = jnp.asarray(np.tile(np.eye(NPG, dtype=np.float32), (1, G_BLK)))
    b1 = mlp_b1.reshape(LAYERS, 1, FILTERS)
    b2 = mlp_b2.reshape(LAYERS, 1, FILTERS)
    cb2 = conv_b2.reshape(LAYERS, 1, HIDDEN)
    ib = int_b.reshape(LAYERS, 1, HIDDEN)
    l1b = lin1_b.reshape(1, HIDDEN // 2)
    l2b = lin2_b.reshape(1, 1)

    def full(shape):
        nd = len(shape)
        return pl.BlockSpec(shape, lambda g, _nd=nd: (0,) * _nd)

    out = pl.pallas_call(
        _body,
        grid=(NP,),
        in_specs=[
            pl.BlockSpec((1, 3, NPG, R), lambda g: (g, 0, 0, 0)),
            pl.BlockSpec((1, 3, R), lambda g: (g, 0, 0)),
            pl.BlockSpec((1, 1, R), lambda g: (g, 0, 0)),
            full((NPG, R)),
            full((100, HIDDEN)),
            full((LAYERS, NG, FILTERS)),
            full((LAYERS, 1, FILTERS)),
            full((LAYERS, FILTERS, FILTERS)),
            full((LAYERS, 1, FILTERS)),
            full((LAYERS, HIDDEN, FILTERS)),
            full((LAYERS, FILTERS, HIDDEN)),
            full((LAYERS, 1, HIDDEN)),
            full((LAYERS, HIDDEN, HIDDEN)),
            full((LAYERS, 1, HIDDEN)),
            full((HIDDEN, HIDDEN // 2)),
            full((1, HIDDEN // 2)),
            full((HIDDEN // 2, 1)),
            full((1, 1)),
        ],
        out_specs=pl.BlockSpec((1, G_BLK, 128), lambda g: (g, 0, 0)),
        out_shape=jax.ShapeDtypeStruct((NP, G_BLK, 128), jnp.float32),
    )(q, prow, zr, eyem, emb, mlp_w1, b1, mlp_w2, b2,
      conv_w1, conv_w2, cb2, int_w, ib, lin1_w, l1b, lin2_w, l2b)
    return out[:, :, 0].reshape(G)


# speedup vs baseline: 4.4957x; 1.2096x over previous
import jax
import jax.numpy as jnp
import numpy as np
from jax.experimental import pallas as pl

N = 10000
G = 100
NPG = 100
K = 16
HIDDEN = 128
FILTERS = 128
LAYERS = 3
NG = 50
CUTOFF = 10.0

G_BLK = 4
R = G_BLK * NPG
E = K * R

_LOG2 = 0.6931471805599453
_STEP = CUTOFF / (NG - 1)
_SQC = (0.5 ** 0.5) / _STEP
_PI = 3.141592653589793
_HI = jax.lax.Precision.HIGHEST
_TL = (((0,), (0,)), ((), ()))


def _sp(x):
    return jnp.maximum(
        x, jnp.log(1.0 + jnp.exp(jnp.minimum(x, 30.0)))) - _LOG2


def _body(q_ref, prow_ref, z_ref, eyem_ref, emb_ref,
          mlp_w1_ref, mlp_b1_ref, mlp_w2_ref, mlp_b2_ref,
          conv_w1_ref, conv_w2_ref, conv_b2_ref,
          int_w_ref, int_b_ref,
          lin1_w_ref, lin1_b_ref, lin2_w_ref, lin2_b_ref,
          out_ref):
    inf = jnp.float32(float("inf"))
    q = q_ref[0]
    pr = prow_ref[0]

    d2 = jnp.zeros((NPG, R), jnp.float32)
    for c in range(3):
        dc = q[c] - pr[c:c + 1, :]
        d2 = d2 + dc * dc
    cur = jnp.where(eyem_ref[...] > 0.0, inf, d2)

    subi = jax.lax.broadcasted_iota(jnp.int32, (NPG, R), 0)

    sel_parts, mv_parts = [], []
    for _ in range(K):
        mv = jnp.min(cur, axis=0, keepdims=True)
        cand = jnp.where(cur <= mv, subi, NPG)
        jmin = jnp.min(cand, axis=0, keepdims=True)
        sel = subi == jmin
        sel_parts.append(jnp.where(sel, 1.0, 0.0))
        mv_parts.append(mv)
        cur = jnp.where(sel, inf, cur)
    D2 = jnp.concatenate(mv_parts, axis=0)

    valid = D2 <= CUTOFF * CUTOFF
    dist = jnp.sqrt(jnp.where(valid, D2, 1.0))
    CV = jnp.where(valid,
                   0.5 * jnp.cos(dist * (_PI / CUTOFF)) + 0.5, 0.0)

    ds = dist * _SQC
    ones_ng = jnp.ones((1, NG), jnp.float32)
    DB = jnp.concatenate(
        [jax.lax.dot_general(ds[k:k + 1], ones_ng, _TL, precision=_HI,
                             preferred_element_type=jnp.float32)
         for k in range(K)], axis=0)
    offs = jax.lax.broadcasted_iota(
        jnp.int32, (E, NG), 1).astype(jnp.float32) * (_STEP * _SQC)
    t = DB - offs
    EA = jnp.exp(-(t * t))

    zr = z_ref[0]
    ohz = jnp.where(subi == zr, 1.0, 0.0)
    h = jax.lax.dot_general(ohz, emb_ref[...], _TL,
                            preferred_element_type=jnp.float32)

    selkb = [[(sel_parts[k] * CV[k:k + 1])[:, b * NPG:(b + 1) * NPG]
              for b in range(G_BLK)] for k in range(K)]

    for l in range(LAYERS):
        A = _sp(jnp.dot(EA, mlp_w1_ref[l],
                        preferred_element_type=jnp.float32) + mlp_b1_ref[l])
        W = jnp.dot(A, mlp_w2_ref[l],
                    preferred_element_type=jnp.float32) + mlp_b2_ref[l]
        hx = jnp.dot(h, conv_w1_ref[l], preferred_element_type=jnp.float32)
        ms = []
        for b in range(G_BLK):
            hxb = hx[b * NPG:(b + 1) * NPG]
            m = None
            for k in range(K):
                XJ = jax.lax.dot_general(
                    selkb[k][b], hxb, _TL,
                    preferred_element_type=jnp.float32)
                p = XJ * W[k * R + b * NPG:k * R + b * NPG + NPG]
                m = p if m is None else m + p
            ms.append(m)
        m = jnp.concatenate(ms, axis=0)
        m = jnp.dot(m, conv_w2_ref[l],
                    preferred_element_type=jnp.float32) + conv_b2_ref[l]
        m = _sp(m)
        m = jnp.dot(m, int_w_ref[l],
                    preferred_element_type=jnp.float32) + int_b_ref[l]
        h = h + m

    t2 = _sp(jnp.dot(h, lin1_w_ref[...],
                     preferred_element_type=jnp.float32) + lin1_b_ref[...])
    y = jnp.dot(t2, lin2_w_ref[...],
                preferred_element_type=jnp.float32) + lin2_b_ref[...]
    outs = jnp.concatenate(
        [jnp.broadcast_to(jnp.sum(y[b * NPG:(b + 1) * NPG]), (1, 128))
         for b in range(G_BLK)], axis=0)
    out_ref[0] = outs


def kernel(z, pos, batch, emb, mlp_w1, mlp_b1, mlp_w2, mlp_b2,
           conv_w1, conv_w2, conv_b2, int_w, int_b,
           lin1_w, lin1_b, lin2_w, lin2_b):
    del batch
    NP = G // G_BLK
    posb = pos.reshape(NP, G_BLK, NPG, 3)
    q = jnp.broadcast_to(
        posb.transpose(0, 3, 2, 1)[:, :, :, :, None],
        (NP, 3, NPG, G_BLK, NPG)).reshape(NP, 3, NPG, R)
    prow = pos.reshape(NP, R, 3).transpose(0, 2, 1)
    zr = z.reshape(NP, 1, R).astype(jnp.int32)
    eyem = jnp.asarray(np.tile(np.eye(NPG, dtype=np.float32), (1, G_BLK)))
    b1 = mlp_b1.reshape(LAYERS, 1, FILTERS)
    b2 = mlp_b2.reshape(LAYERS, 1, FILTERS)
    cb2 = conv_b2.reshape(LAYERS, 1, HIDDEN)
    ib = int_b.reshape(LAYERS, 1, HIDDEN)
    l1b = lin1_b.reshape(1, HIDDEN // 2)
    l2b = lin2_b.reshape(1, 1)

    def full(shape):
        nd = len(shape)
        return pl.BlockSpec(shape, lambda g, _nd=nd: (0,) * _nd)

    out = pl.pallas_call(
        _body,
        grid=(NP,),
        in_specs=[
            pl.BlockSpec((1, 3, NPG, R), lambda g: (g, 0, 0, 0)),
            pl.BlockSpec((1, 3, R), lambda g: (g, 0, 0)),
            pl.BlockSpec((1, 1, R), lambda g: (g, 0, 0)),
            full((NPG, R)),
            full((100, HIDDEN)),
            full((LAYERS, NG, FILTERS)),
            full((LAYERS, 1, FILTERS)),
            full((LAYERS, FILTERS, FILTERS)),
            full((LAYERS, 1, FILTERS)),
            full((LAYERS, HIDDEN, FILTERS)),
            full((LAYERS, FILTERS, HIDDEN)),
            full((LAYERS, 1, HIDDEN)),
            full((LAYERS, HIDDEN, HIDDEN)),
            full((LAYERS, 1, HIDDEN)),
            full((HIDDEN, HIDDEN // 2)),
            full((1, HIDDEN // 2)),
            full((HIDDEN // 2, 1)),
            full((1, 1)),
        ],
        out_specs=pl.BlockSpec((1, G_BLK, 128), lambda g: (g, 0, 0)),
        out_shape=jax.ShapeDtypeStruct((NP, G_BLK, 128), jnp.float32),
    )(q, prow, zr, eyem, emb, mlp_w1, b1, mlp_w2, b2,
      conv_w1, conv_w2, cb2, int_w, ib, lin1_w, l1b, lin2_w, l2b)
    return out[:, :, 0].reshape(G)


# speedup vs baseline: 4.9234x; 1.0951x over previous
import jax
import jax.numpy as jnp
import numpy as np
from jax.experimental import pallas as pl

N = 10000
G = 100
NPG = 100
K = 16
HIDDEN = 128
FILTERS = 128
LAYERS = 3
NG = 50
CUTOFF = 10.0

G_BLK = 4
R = G_BLK * NPG
E = K * R

_LOG2 = 0.6931471805599453
_STEP = CUTOFF / (NG - 1)
_SQC = (0.5 ** 0.5) / _STEP
_PI = 3.141592653589793
_HI = jax.lax.Precision.HIGHEST
_TL = (((0,), (0,)), ((), ()))


def _sp(x):
    return jnp.maximum(
        x, jnp.log(1.0 + jnp.exp(jnp.minimum(x, 30.0)))) - _LOG2


def _body(q_ref, prow_ref, z_ref, eyem_ref, emb_ref,
          mlp_w1_ref, mlp_b1_ref, mlp_w2_ref, mlp_b2_ref,
          conv_w1_ref, conv_w2_ref, conv_b2_ref,
          int_w_ref, int_b_ref,
          lin1_w_ref, lin1_b_ref, lin2_w_ref, lin2_b_ref,
          out_ref):
    inf = jnp.float32(float("inf"))
    q = q_ref[0]
    pr = prow_ref[0]

    d2 = jnp.zeros((NPG, R), jnp.float32)
    for c in range(3):
        dc = q[c] - pr[c:c + 1, :]
        d2 = d2 + dc * dc
    cur = jnp.where(eyem_ref[...] > 0.0, inf, d2)

    subi = jax.lax.broadcasted_iota(jnp.int32, (NPG, R), 0)

    sel_parts, mv_parts = [], []
    for _ in range(K):
        mv = jnp.min(cur, axis=0, keepdims=True)
        cand = jnp.where(cur <= mv, subi, NPG)
        jmin = jnp.min(cand, axis=0, keepdims=True)
        sel = subi == jmin
        sel_parts.append(jnp.where(sel, 1.0, 0.0))
        mv_parts.append(mv)
        cur = jnp.where(sel, inf, cur)
    D2 = jnp.concatenate(mv_parts, axis=0)

    valid = D2 <= CUTOFF * CUTOFF
    dist = jnp.sqrt(jnp.where(valid, D2, 1.0))
    CV = jnp.where(valid,
                   0.5 * jnp.cos(dist * (_PI / CUTOFF)) + 0.5, 0.0)

    ds = dist * _SQC
    dsrow = jnp.concatenate([ds[k:k + 1] for k in range(K)], axis=1)
    offs = jax.lax.broadcasted_iota(
        jnp.int32, (NG, E), 0).astype(jnp.float32) * (_STEP * _SQC)
    t = dsrow - offs
    EAT = jnp.exp(-(t * t))

    zr = z_ref[0]
    ohz = jnp.where(subi == zr, 1.0, 0.0)
    h = jax.lax.dot_general(ohz, emb_ref[...], _TL,
                            preferred_element_type=jnp.float32)

    selkb = [[(sel_parts[k] * CV[k:k + 1])[:, b * NPG:(b + 1) * NPG]
              for b in range(G_BLK)] for k in range(K)]

    for l in range(LAYERS):
        A = _sp(jax.lax.dot_general(EAT, mlp_w1_ref[l], _TL,
                                    preferred_element_type=jnp.float32)
                + mlp_b1_ref[l])
        W = jnp.dot(A, mlp_w2_ref[l],
                    preferred_element_type=jnp.float32) + mlp_b2_ref[l]
        hx = jnp.dot(h, conv_w1_ref[l], preferred_element_type=jnp.float32)
        ms = []
        for b in range(G_BLK):
            hxb = hx[b * NPG:(b + 1) * NPG]
            m = None
            for k in range(K):
                XJ = jax.lax.dot_general(
                    selkb[k][b], hxb, _TL,
                    preferred_element_type=jnp.float32)
                p = XJ * W[k * R + b * NPG:k * R + b * NPG + NPG]
                m = p if m is None else m + p
            ms.append(m)
        m = jnp.concatenate(ms, axis=0)
        m = jnp.dot(m, conv_w2_ref[l],
                    preferred_element_type=jnp.float32) + conv_b2_ref[l]
        m = _sp(m)
        m = jnp.dot(m, int_w_ref[l],
                    preferred_element_type=jnp.float32) + int_b_ref[l]
        h = h + m

    t2 = _sp(jnp.dot(h, lin1_w_ref[...],
                     preferred_element_type=jnp.float32) + lin1_b_ref[...])
    y = jnp.dot(t2, lin2_w_ref[...],
                preferred_element_type=jnp.float32) + lin2_b_ref[...]
    outs = jnp.concatenate(
        [jnp.broadcast_to(jnp.sum(y[b * NPG:(b + 1) * NPG]), (1, 128))
         for b in range(G_BLK)], axis=0)
    out_ref[0] = outs


def kernel(z, pos, batch, emb, mlp_w1, mlp_b1, mlp_w2, mlp_b2,
           conv_w1, conv_w2, conv_b2, int_w, int_b,
           lin1_w, lin1_b, lin2_w, lin2_b):
    del batch
    NP = G // G_BLK
    posb = pos.reshape(NP, G_BLK, NPG, 3)
    q = jnp.broadcast_to(
        posb.transpose(0, 3, 2, 1)[:, :, :, :, None],
        (NP, 3, NPG, G_BLK, NPG)).reshape(NP, 3, NPG, R)
    prow = pos.reshape(NP, R, 3).transpose(0, 2, 1)
    zr = z.reshape(NP, 1, R).astype(jnp.int32)
    eyem = jnp.asarray(np.tile(np.eye(NPG, dtype=np.float32), (1, G_BLK)))
    b1 = mlp_b1.reshape(LAYERS, 1, FILTERS)
    b2 = mlp_b2.reshape(LAYERS, 1, FILTERS)
    cb2 = conv_b2.reshape(LAYERS, 1, HIDDEN)
    ib = int_b.reshape(LAYERS, 1, HIDDEN)
    l1b = lin1_b.reshape(1, HIDDEN // 2)
    l2b = lin2_b.reshape(1, 1)

    def full(shape):
        nd = len(shape)
        return pl.BlockSpec(shape, lambda g, _nd=nd: (0,) * _nd)

    out = pl.pallas_call(
        _body,
        grid=(NP,),
        in_specs=[
            pl.BlockSpec((1, 3, NPG, R), lambda g: (g, 0, 0, 0)),
            pl.BlockSpec((1, 3, R), lambda g: (g, 0, 0)),
            pl.BlockSpec((1, 1, R), lambda g: (g, 0, 0)),
            full((NPG, R)),
            full((100, HIDDEN)),
            full((LAYERS, NG, FILTERS)),
            full((LAYERS, 1, FILTERS)),
            full((LAYERS, FILTERS, FILTERS)),
            full((LAYERS, 1, FILTERS)),
            full((LAYERS, HIDDEN, FILTERS)),
            full((LAYERS, FILTERS, HIDDEN)),
            full((LAYERS, 1, HIDDEN)),
            full((LAYERS, HIDDEN, HIDDEN)),
            full((LAYERS, 1, HIDDEN)),
            full((HIDDEN, HIDDEN // 2)),
            full((1, HIDDEN // 2)),
            full((HIDDEN // 2, 1)),
            full((1, 1)),
        ],
        out_specs=pl.BlockSpec((1, G_BLK, 128), lambda g: (g, 0, 0)),
        out_shape=jax.ShapeDtypeStruct((NP, G_BLK, 128), jnp.float32),
    )(q, prow, zr, eyem, emb, mlp_w1, b1, mlp_w2, b2,
      conv_w1, conv_w2, cb2, int_w, ib, lin1_w, l1b, lin2_w, l2b)
    return out[:, :, 0].reshape(G)
